# baseline (speedup 1.0000x reference)
import functools

import jax
import jax.numpy as jnp
from jax import lax
from jax.experimental import pallas as pl
from jax.experimental.pallas import tpu as pltpu
from jax.experimental.pallas import tpu_sc as plsc

VOCAB_SIZE = 100000
N_COMMS = 64
ALPHA = 0.01
SEQ_LEN = 200
BATCH = 1024

_NC = 2
_NS = 16
_NW = _NC * _NS
_LANES = 16

_NG = BATCH // _LANES
_SCALE = 1024.0

_NWB = 4
_WB = 25088
_VW = _NWB * _WB


def _tab_body(freq_ref, n_ref, m_ref, lens_ref, out_ref, mm_ref):
    i = pl.program_id(0)
    col_ok = i * _WB + lax.broadcasted_iota(jnp.int32, (_NW, _WB), 1) < VOCAB_SIZE
    n = n_ref[...]
    ld = jnp.log(n + VOCAB_SIZE * ALPHA)
    p = freq_ref[...] * n
    p = jnp.where(p == 0.0, ALPHA, p)
    lp = jnp.log(p) - ld
    q = jnp.floor(lp * _SCALE + 0.5).astype(jnp.int32)
    q = jnp.clip(q, -32768, 32767)
    lo = jnp.where(col_ok, q[: _NW, :], 0)
    hi = jnp.where(col_ok, q[_NW :, :], 0)
    out_ref[...] = (lo & 0xFFFF) | (hi << 16)

    @pl.when(i == 0)
    def _():
        pos = lax.broadcasted_iota(jnp.int32, (SEQ_LEN, BATCH), 0)
        mm_ref[...] = jnp.where(pos < lens_ref[0, :][None, :], m_ref[...],
                                VOCAB_SIZE)


_tab_call = pl.pallas_call(
    _tab_body,
    grid=(_NWB,),
    in_specs=[
        pl.BlockSpec((N_COMMS, _WB), lambda i: (0, i)),
        pl.BlockSpec((N_COMMS, 1), lambda i: (0, 0)),
        pl.BlockSpec((SEQ_LEN, BATCH), lambda i: (0, 0)),
        pl.BlockSpec((1, BATCH), lambda i: (0, 0)),
    ],
    out_specs=[
        pl.BlockSpec((_NW, _WB), lambda i: (0, i)),
        pl.BlockSpec((SEQ_LEN, BATCH), lambda i: (0, 0)),
    ],
    out_shape=[
        jax.ShapeDtypeStruct((_NW, _VW), jnp.int32),
        jax.ShapeDtypeStruct((SEQ_LEN, BATCH), jnp.int32),
    ],
)


def _smax_body(nll_ref, out_ref):
    x = nll_ref[...].astype(jnp.float32) * (1.0 / _SCALE)
    e = jnp.exp(x - jnp.max(x, axis=0, keepdims=True))
    out_ref[...] = (e / jnp.sum(e, axis=0, keepdims=True)).T


_smax_call = pl.pallas_call(
    _smax_body,
    out_shape=jax.ShapeDtypeStruct((BATCH, N_COMMS), jnp.float32),
)


_RC = 10
_NCH = SEQ_LEN // _RC


def _sc_body(tab_hbm, mm_hbm, out_hbm, shard_v, ib0, ib1, out_v, sem0, sem1):
    t = lax.axis_index("s") * _NC + lax.axis_index("c")
    pltpu.sync_copy(tab_hbm.at[t], shard_v)

    z = jnp.zeros((_LANES,), jnp.int32)

    @pl.loop(0, _NG)
    def _zero(g):
        out_v[0, pl.ds(g * _LANES, _LANES)] = z
        out_v[1, pl.ds(g * _LANES, _LANES)] = z

    ibs = (ib0, ib1)
    sems = (sem0, sem1)

    def _fire(c, slot):
        pltpu.async_copy(mm_hbm.at[pl.ds(c * _RC, _RC), :], ibs[slot],
                         sems[slot])

    def _process(c, slot):
        ib = ibs[slot]
        pltpu.make_async_copy(mm_hbm.at[pl.ds(c * _RC, _RC), :], ib,
                              sems[slot]).wait()

        @pl.loop(0, _NG, unroll=4)
        def _grp(g):
            a0, a1 = z, z
            for r in range(_RC):
                pv = plsc.load_gather(shard_v, [ib[r, pl.ds(g * _LANES, _LANES)]])
                a0 = a0 + ((pv << 16) >> 16)
                a1 = a1 + (pv >> 16)
            plsc.addupdate(out_v.at[0, pl.ds(g * _LANES, _LANES)], a0)
            plsc.addupdate(out_v.at[1, pl.ds(g * _LANES, _LANES)], a1)

    _fire(0, 0)

    @pl.loop(0, _NCH, step=2)
    def _chunk2(c0):
        _fire(c0 + 1, 1)
        _process(c0, 0)

        @pl.when(c0 + 2 < _NCH)
        def _():
            _fire(c0 + 2, 0)

        _process(c0 + 1, 1)

    pltpu.sync_copy(out_v.at[0], out_hbm.at[t])
    pltpu.sync_copy(out_v.at[1], out_hbm.at[t + _NW])


@functools.cache
def _make_sc_call():
    return functools.partial(
        pl.kernel,
        out_type=jax.ShapeDtypeStruct((N_COMMS, BATCH), jnp.int32),
        mesh=plsc.VectorSubcoreMesh(
            core_axis_name="c", subcore_axis_name="s", num_cores=_NC, num_subcores=_NS
        ),
        compiler_params=pltpu.CompilerParams(
            needs_layout_passes=False, use_tc_tiling_on_sc=False
        ),
        scratch_types=[
            pltpu.VMEM((_VW,), jnp.int32),
            pltpu.VMEM((_RC, BATCH), jnp.int32),
            pltpu.VMEM((_RC, BATCH), jnp.int32),
            pltpu.VMEM((2, BATCH), jnp.int32),
            pltpu.SemaphoreType.DMA,
            pltpu.SemaphoreType.DMA,
        ],
    )(_sc_body)


def kernel(m, m_lens, unigram_freq, comm_N):
    packed, mm = _tab_call(unigram_freq, comm_N.reshape(N_COMMS, 1), m,
                           m_lens.reshape(1, BATCH))
    nll = _make_sc_call()(packed, mm)
    return _smax_call(nll)


# speedup vs baseline: 1.0659x; 1.0659x over previous
import functools

import jax
import jax.numpy as jnp
from jax import lax
from jax.experimental import pallas as pl
from jax.experimental.pallas import tpu as pltpu
from jax.experimental.pallas import tpu_sc as plsc

VOCAB_SIZE = 100000
N_COMMS = 64
ALPHA = 0.01
SEQ_LEN = 200
BATCH = 1024

_NC = 2
_NS = 16
_NW = _NC * _NS
_LANES = 16

_NG = BATCH // _LANES
_SCALE = 1024.0

_VW = 100096
_BC = 8
_NCB = _NW // _BC


def _tab_body(freq_lo_ref, freq_hi_ref, n_ref, m_ref, lens_ref, out_ref,
              mm_ref):
    i = pl.program_id(0)
    col_ok = lax.broadcasted_iota(jnp.int32, (_BC, _VW), 1) < VOCAB_SIZE
    qs = []
    for half, f_ref in ((0, freq_lo_ref), (1, freq_hi_ref)):
        nn = n_ref[pl.ds(i * _BC + half * _NW, _BC), 0]
        ld = jnp.log(nn + VOCAB_SIZE * ALPHA)
        p = f_ref[...] * nn[:, None]
        p = jnp.where(p == 0.0, ALPHA, p)
        lp = jnp.log(p) - ld[:, None]
        q = jnp.floor(lp * _SCALE + 0.5).astype(jnp.int32)
        q = jnp.clip(q, -32768, 32767)
        qs.append(jnp.where(col_ok, q, 0))
    out_ref[...] = (qs[0] & 0xFFFF) | (qs[1] << 16)

    @pl.when(i == 0)
    def _():
        pos = lax.broadcasted_iota(jnp.int32, (SEQ_LEN, BATCH), 0)
        mm_ref[...] = jnp.where(pos < lens_ref[0, :][None, :], m_ref[...],
                                VOCAB_SIZE)


_tab_call = pl.pallas_call(
    _tab_body,
    grid=(_NCB,),
    in_specs=[
        pl.BlockSpec((_BC, _VW), lambda i: (i, 0)),
        pl.BlockSpec((_BC, _VW), lambda i: (i + _NCB, 0)),
        pl.BlockSpec((N_COMMS, 1), lambda i: (0, 0)),
        pl.BlockSpec((SEQ_LEN, BATCH), lambda i: (0, 0)),
        pl.BlockSpec((1, BATCH), lambda i: (0, 0)),
    ],
    out_specs=[
        pl.BlockSpec((_BC, _VW), lambda i: (i, 0)),
        pl.BlockSpec((SEQ_LEN, BATCH), lambda i: (0, 0)),
    ],
    out_shape=[
        jax.ShapeDtypeStruct((_NW, _VW), jnp.int32),
        jax.ShapeDtypeStruct((SEQ_LEN, BATCH), jnp.int32),
    ],
)


def _smax_body(nll_ref, out_ref):
    x = nll_ref[...].astype(jnp.float32) * (1.0 / _SCALE)
    e = jnp.exp(x - jnp.max(x, axis=0, keepdims=True))
    out_ref[...] = (e / jnp.sum(e, axis=0, keepdims=True)).T


_smax_call = pl.pallas_call(
    _smax_body,
    out_shape=jax.ShapeDtypeStruct((BATCH, N_COMMS), jnp.float32),
)


_RC = 10
_NCH = SEQ_LEN // _RC


def _sc_body(tab_hbm, mm_hbm, out_hbm, shard_v, ib0, ib1, out_v, sem0, sem1):
    t = lax.axis_index("s") * _NC + lax.axis_index("c")
    pltpu.sync_copy(tab_hbm.at[t], shard_v)

    z = jnp.zeros((_LANES,), jnp.int32)

    @pl.loop(0, _NG)
    def _zero(g):
        out_v[0, pl.ds(g * _LANES, _LANES)] = z
        out_v[1, pl.ds(g * _LANES, _LANES)] = z

    ibs = (ib0, ib1)
    sems = (sem0, sem1)

    def _fire(c, slot):
        pltpu.async_copy(mm_hbm.at[pl.ds(c * _RC, _RC), :], ibs[slot],
                         sems[slot])

    def _process(c, slot):
        ib = ibs[slot]
        pltpu.make_async_copy(mm_hbm.at[pl.ds(c * _RC, _RC), :], ib,
                              sems[slot]).wait()

        @pl.loop(0, _NG, unroll=4)
        def _grp(g):
            a0, a1 = z, z
            for r in range(_RC):
                pv = plsc.load_gather(shard_v, [ib[r, pl.ds(g * _LANES, _LANES)]])
                a0 = a0 + ((pv << 16) >> 16)
                a1 = a1 + (pv >> 16)
            plsc.addupdate(out_v.at[0, pl.ds(g * _LANES, _LANES)], a0)
            plsc.addupdate(out_v.at[1, pl.ds(g * _LANES, _LANES)], a1)

    _fire(0, 0)

    @pl.loop(0, _NCH, step=2)
    def _chunk2(c0):
        _fire(c0 + 1, 1)
        _process(c0, 0)

        @pl.when(c0 + 2 < _NCH)
        def _():
            _fire(c0 + 2, 0)

        _process(c0 + 1, 1)

    pltpu.sync_copy(out_v.at[0], out_hbm.at[t])
    pltpu.sync_copy(out_v.at[1], out_hbm.at[t + _NW])


@functools.cache
def _make_sc_call():
    return functools.partial(
        pl.kernel,
        out_type=jax.ShapeDtypeStruct((N_COMMS, BATCH), jnp.int32),
        mesh=plsc.VectorSubcoreMesh(
            core_axis_name="c", subcore_axis_name="s", num_cores=_NC, num_subcores=_NS
        ),
        compiler_params=pltpu.CompilerParams(
            needs_layout_passes=False, use_tc_tiling_on_sc=False
        ),
        scratch_types=[
            pltpu.VMEM((_VW,), jnp.int32),
            pltpu.VMEM((_RC, BATCH), jnp.int32),
            pltpu.VMEM((_RC, BATCH), jnp.int32),
            pltpu.VMEM((2, BATCH), jnp.int32),
            pltpu.SemaphoreType.DMA,
            pltpu.SemaphoreType.DMA,
        ],
    )(_sc_body)


def kernel(m, m_lens, unigram_freq, comm_N):
    packed, mm = _tab_call(unigram_freq, unigram_freq,
                           comm_N.reshape(N_COMMS, 1), m,
                           m_lens.reshape(1, BATCH))
    nll = _make_sc_call()(packed, mm)
    return _smax_call(nll)


# speedup vs baseline: 1.0826x; 1.0156x over previous
import functools

import jax
import jax.numpy as jnp
from jax import lax
from jax.experimental import pallas as pl
from jax.experimental.pallas import tpu as pltpu
from jax.experimental.pallas import tpu_sc as plsc

VOCAB_SIZE = 100000
N_COMMS = 64
ALPHA = 0.01
SEQ_LEN = 200
BATCH = 1024

_NC = 2
_NS = 16
_NW = _NC * _NS
_LANES = 16

_NG = BATCH // _LANES
_SCALE = 1024.0

_VW = 100096
_BC = 8
_NCB = _NW // _BC


def _tab_body(freq_lo_ref, freq_hi_ref, n_ref, m_ref, lens_ref, out_ref,
              mm_ref):
    i = pl.program_id(0)
    col_ok = lax.broadcasted_iota(jnp.int32, (_BC, _VW), 1) < VOCAB_SIZE
    qs = []
    for half, f_ref in ((0, freq_lo_ref), (1, freq_hi_ref)):
        nn = n_ref[pl.ds(i * _BC + half * _NW, _BC), 0]
        ld = jnp.log(nn + VOCAB_SIZE * ALPHA)
        p = f_ref[...] * nn[:, None]
        p = jnp.where(p == 0.0, ALPHA, p)
        lp = jnp.log(p) - ld[:, None]
        q = jnp.floor(lp * _SCALE + 0.5).astype(jnp.int32)
        q = jnp.clip(q, -32768, 32767)
        qs.append(jnp.where(col_ok, q, 0))
    out_ref[...] = (qs[0] & 0xFFFF) | (qs[1] << 16)

    @pl.when(i == 0)
    def _():
        pos = lax.broadcasted_iota(jnp.int32, (SEQ_LEN, BATCH), 0)
        mm_ref[...] = jnp.where(pos < lens_ref[0, :][None, :], m_ref[...],
                                VOCAB_SIZE)


_tab_call = pl.pallas_call(
    _tab_body,
    grid=(_NCB,),
    in_specs=[
        pl.BlockSpec((_BC, _VW), lambda i: (i, 0)),
        pl.BlockSpec((_BC, _VW), lambda i: (i + _NCB, 0)),
        pl.BlockSpec((N_COMMS, 1), lambda i: (0, 0)),
        pl.BlockSpec((SEQ_LEN, BATCH), lambda i: (0, 0)),
        pl.BlockSpec((1, BATCH), lambda i: (0, 0)),
    ],
    out_specs=[
        pl.BlockSpec((_BC, _VW), lambda i: (i, 0)),
        pl.BlockSpec((SEQ_LEN, BATCH), lambda i: (0, 0)),
    ],
    out_shape=[
        jax.ShapeDtypeStruct((_NW, _VW), jnp.int32),
        jax.ShapeDtypeStruct((SEQ_LEN, BATCH), jnp.int32),
    ],
)


def _smax_body(nll_ref, out_ref):
    x = nll_ref[...].astype(jnp.float32) * (1.0 / _SCALE)
    e = jnp.exp(x - jnp.max(x, axis=0, keepdims=True))
    out_ref[...] = (e / jnp.sum(e, axis=0, keepdims=True)).T


_smax_call = pl.pallas_call(
    _smax_body,
    out_shape=jax.ShapeDtypeStruct((BATCH, N_COMMS), jnp.float32),
)


_RC = 10
_NCH = SEQ_LEN // _RC


def _sc_body(tab_hbm, mm_hbm, out_hbm, shard_v, ib0, ib1, out_v, sem0, sem1):
    t = lax.axis_index("s") * _NC + lax.axis_index("c")
    pltpu.sync_copy(tab_hbm.at[t], shard_v)

    z = jnp.zeros((_LANES,), jnp.int32)

    @pl.loop(0, _NG)
    def _zero(g):
        out_v[0, pl.ds(g * _LANES, _LANES)] = z
        out_v[1, pl.ds(g * _LANES, _LANES)] = z

    ibs = (ib0, ib1)
    sems = (sem0, sem1)

    def _fire(c, slot):
        pltpu.async_copy(mm_hbm.at[pl.ds(c * _RC, _RC), :], ibs[slot],
                         sems[slot])

    def _process(c, slot):
        ib = ibs[slot]
        pltpu.make_async_copy(mm_hbm.at[pl.ds(c * _RC, _RC), :], ib,
                              sems[slot]).wait()

        @pl.loop(0, _NG, unroll=8)
        def _grp(g):
            a0, a1 = z, z
            for r in range(_RC):
                pv = plsc.load_gather(shard_v, [ib[r, pl.ds(g * _LANES, _LANES)]])
                a0 = a0 + ((pv << 16) >> 16)
                a1 = a1 + (pv >> 16)
            plsc.addupdate(out_v.at[0, pl.ds(g * _LANES, _LANES)], a0)
            plsc.addupdate(out_v.at[1, pl.ds(g * _LANES, _LANES)], a1)

    _fire(0, 0)

    @pl.loop(0, _NCH, step=2)
    def _chunk2(c0):
        _fire(c0 + 1, 1)
        _process(c0, 0)

        @pl.when(c0 + 2 < _NCH)
        def _():
            _fire(c0 + 2, 0)

        _process(c0 + 1, 1)

    pltpu.sync_copy(out_v.at[0], out_hbm.at[t])
    pltpu.sync_copy(out_v.at[1], out_hbm.at[t + _NW])


@functools.cache
def _make_sc_call():
    return functools.partial(
        pl.kernel,
        out_type=jax.ShapeDtypeStruct((N_COMMS, BATCH), jnp.int32),
        mesh=plsc.VectorSubcoreMesh(
            core_axis_name="c", subcore_axis_name="s", num_cores=_NC, num_subcores=_NS
        ),
        compiler_params=pltpu.CompilerParams(
            needs_layout_passes=False, use_tc_tiling_on_sc=False
        ),
        scratch_types=[
            pltpu.VMEM((_VW,), jnp.int32),
            pltpu.VMEM((_RC, BATCH), jnp.int32),
            pltpu.VMEM((_RC, BATCH), jnp.int32),
            pltpu.VMEM((2, BATCH), jnp.int32),
            pltpu.SemaphoreType.DMA,
            pltpu.SemaphoreType.DMA,
        ],
    )(_sc_body)


def kernel(m, m_lens, unigram_freq, comm_N):
    packed, mm = _tab_call(unigram_freq, unigram_freq,
                           comm_N.reshape(N_COMMS, 1), m,
                           m_lens.reshape(1, BATCH))
    nll = _make_sc_call()(packed, mm)
    return _smax_call(nll)


# speedup vs baseline: 1.0985x; 1.0147x over previous
import functools

import jax
import jax.numpy as jnp
from jax import lax
from jax.experimental import pallas as pl
from jax.experimental.pallas import tpu as pltpu
from jax.experimental.pallas import tpu_sc as plsc

VOCAB_SIZE = 100000
N_COMMS = 64
ALPHA = 0.01
SEQ_LEN = 200
BATCH = 1024

_NC = 2
_NS = 16
_NW = _NC * _NS
_LANES = 16

_NG = BATCH // _LANES
_SCALE = 1024.0

_VW = 100096
_BC = 8
_NCB = _NW // _BC


def _tab_body(freq_lo_ref, freq_hi_ref, n_ref, m_ref, lens_ref, out_ref,
              mm_ref):
    i = pl.program_id(0)
    col_ok = lax.broadcasted_iota(jnp.int32, (_BC, _VW), 1) < VOCAB_SIZE
    qs = []
    for half, f_ref in ((0, freq_lo_ref), (1, freq_hi_ref)):
        nn = n_ref[pl.ds(i * _BC + half * _NW, _BC), 0]
        ld = jnp.log(nn + VOCAB_SIZE * ALPHA)
        p = f_ref[...] * nn[:, None]
        p = jnp.where(p == 0.0, ALPHA, p)
        lp = jnp.log(p) - ld[:, None]
        q = jnp.floor(lp * _SCALE + 0.5).astype(jnp.int32)
        q = jnp.clip(q, -32768, 32767)
        qs.append(jnp.where(col_ok, q, 0))
    out_ref[...] = (qs[0] & 0xFFFF) | (qs[1] << 16)

    @pl.when(i == 0)
    def _():
        pos = lax.broadcasted_iota(jnp.int32, (SEQ_LEN, BATCH), 0)
        mm_ref[...] = jnp.where(pos < lens_ref[0, :][None, :], m_ref[...],
                                VOCAB_SIZE)


_tab_call = pl.pallas_call(
    _tab_body,
    grid=(_NCB,),
    in_specs=[
        pl.BlockSpec((_BC, _VW), lambda i: (i, 0)),
        pl.BlockSpec((_BC, _VW), lambda i: (i + _NCB, 0)),
        pl.BlockSpec((N_COMMS, 1), lambda i: (0, 0)),
        pl.BlockSpec((SEQ_LEN, BATCH), lambda i: (0, 0)),
        pl.BlockSpec((1, BATCH), lambda i: (0, 0)),
    ],
    out_specs=[
        pl.BlockSpec((_BC, _VW), lambda i: (i, 0)),
        pl.BlockSpec((SEQ_LEN, BATCH), lambda i: (0, 0)),
    ],
    out_shape=[
        jax.ShapeDtypeStruct((_NW, _VW), jnp.int32),
        jax.ShapeDtypeStruct((SEQ_LEN, BATCH), jnp.int32),
    ],
)


def _smax_body(nll_ref, out_ref):
    x = nll_ref[...].astype(jnp.float32) * (1.0 / _SCALE)
    e = jnp.exp(x - jnp.max(x, axis=0, keepdims=True))
    out_ref[...] = (e / jnp.sum(e, axis=0, keepdims=True)).T


_smax_call = pl.pallas_call(
    _smax_body,
    out_shape=jax.ShapeDtypeStruct((BATCH, N_COMMS), jnp.float32),
)


_RC = 10
_NCH = SEQ_LEN // _RC


def _sc_body(tab_hbm, mm_hbm, out_hbm, shard_v, ib0, ib1, out_v, sem0, sem1):
    t = lax.axis_index("s") * _NC + lax.axis_index("c")
    pltpu.sync_copy(tab_hbm.at[t], shard_v)

    z = jnp.zeros((_LANES,), jnp.int32)

    @pl.loop(0, _NG)
    def _zero(g):
        out_v[0, pl.ds(g * _LANES, _LANES)] = z
        out_v[1, pl.ds(g * _LANES, _LANES)] = z

    ibs = (ib0, ib1)
    sems = (sem0, sem1)

    def _fire(c, slot):
        pltpu.async_copy(mm_hbm.at[pl.ds(c * _RC, _RC), :], ibs[slot],
                         sems[slot])

    def _process(c, slot):
        ib = ibs[slot]
        pltpu.make_async_copy(mm_hbm.at[pl.ds(c * _RC, _RC), :], ib,
                              sems[slot]).wait()

        @plsc.parallel_loop(0, _NG, unroll=8)
        def _grp(g):
            a0, a1 = z, z
            for r in range(_RC):
                pv = plsc.load_gather(shard_v, [ib[r, pl.ds(g * _LANES, _LANES)]])
                a0 = a0 + ((pv << 16) >> 16)
                a1 = a1 + (pv >> 16)
            plsc.addupdate(out_v.at[0, pl.ds(g * _LANES, _LANES)], a0)
            plsc.addupdate(out_v.at[1, pl.ds(g * _LANES, _LANES)], a1)

    _fire(0, 0)

    @pl.loop(0, _NCH, step=2)
    def _chunk2(c0):
        _fire(c0 + 1, 1)
        _process(c0, 0)

        @pl.when(c0 + 2 < _NCH)
        def _():
            _fire(c0 + 2, 0)

        _process(c0 + 1, 1)

    pltpu.sync_copy(out_v.at[0], out_hbm.at[t])
    pltpu.sync_copy(out_v.at[1], out_hbm.at[t + _NW])


@functools.cache
def _make_sc_call():
    return functools.partial(
        pl.kernel,
        out_type=jax.ShapeDtypeStruct((N_COMMS, BATCH), jnp.int32),
        mesh=plsc.VectorSubcoreMesh(
            core_axis_name="c", subcore_axis_name="s", num_cores=_NC, num_subcores=_NS
        ),
        compiler_params=pltpu.CompilerParams(
            needs_layout_passes=False, use_tc_tiling_on_sc=False
        ),
        scratch_types=[
            pltpu.VMEM((_VW,), jnp.int32),
            pltpu.VMEM((_RC, BATCH), jnp.int32),
            pltpu.VMEM((_RC, BATCH), jnp.int32),
            pltpu.VMEM((2, BATCH), jnp.int32),
            pltpu.SemaphoreType.DMA,
            pltpu.SemaphoreType.DMA,
        ],
    )(_sc_body)


def kernel(m, m_lens, unigram_freq, comm_N):
    packed, mm = _tab_call(unigram_freq, unigram_freq,
                           comm_N.reshape(N_COMMS, 1), m,
                           m_lens.reshape(1, BATCH))
    nll = _make_sc_call()(packed, mm)
    return _smax_call(nll)


# speedup vs baseline: 1.1097x; 1.0103x over previous
import functools

import jax
import jax.numpy as jnp
from jax import lax
from jax.experimental import pallas as pl
from jax.experimental.pallas import tpu as pltpu
from jax.experimental.pallas import tpu_sc as plsc

VOCAB_SIZE = 100000
N_COMMS = 64
ALPHA = 0.01
SEQ_LEN = 200
BATCH = 1024

_NC = 2
_NS = 16
_NW = _NC * _NS
_LANES = 16

_NG = BATCH // _LANES
_SCALE = 1024.0

_VW = 100096
_BC = 8
_NCB = _NW // _BC


def _tab_body(freq_lo_ref, freq_hi_ref, n_ref, m_ref, lens_ref, out_ref,
              mm_ref):
    i = pl.program_id(0)
    col_ok = lax.broadcasted_iota(jnp.int32, (_BC, _VW), 1) < VOCAB_SIZE
    qs = []
    for half, f_ref in ((0, freq_lo_ref), (1, freq_hi_ref)):
        nn = n_ref[pl.ds(i * _BC + half * _NW, _BC), 0]
        ld = jnp.log(nn + VOCAB_SIZE * ALPHA)
        p = f_ref[...] * nn[:, None]
        p = jnp.where(p == 0.0, ALPHA, p)
        lp = jnp.log(p) - ld[:, None]
        q = jnp.floor(lp * _SCALE + 0.5).astype(jnp.int32)
        q = jnp.clip(q, -32768, 32767)
        qs.append(jnp.where(col_ok, q, 0))
    out_ref[...] = (qs[0] & 0xFFFF) | (qs[1] << 16)

    @pl.when(i == 0)
    def _():
        pos = lax.broadcasted_iota(jnp.int32, (SEQ_LEN, BATCH), 0)
        mm_ref[...] = jnp.where(pos < lens_ref[0, :][None, :], m_ref[...],
                                VOCAB_SIZE)


_tab_call = pl.pallas_call(
    _tab_body,
    grid=(_NCB,),
    in_specs=[
        pl.BlockSpec((_BC, _VW), lambda i: (i, 0)),
        pl.BlockSpec((_BC, _VW), lambda i: (i + _NCB, 0)),
        pl.BlockSpec((N_COMMS, 1), lambda i: (0, 0)),
        pl.BlockSpec((SEQ_LEN, BATCH), lambda i: (0, 0)),
        pl.BlockSpec((1, BATCH), lambda i: (0, 0)),
    ],
    out_specs=[
        pl.BlockSpec((_BC, _VW), lambda i: (i, 0)),
        pl.BlockSpec((SEQ_LEN, BATCH), lambda i: (0, 0)),
    ],
    out_shape=[
        jax.ShapeDtypeStruct((_NW, _VW), jnp.int32),
        jax.ShapeDtypeStruct((SEQ_LEN, BATCH), jnp.int32),
    ],
)


def _smax_body(nll_ref, out_ref):
    x = nll_ref[...].astype(jnp.float32) * (1.0 / _SCALE)
    e = jnp.exp(x - jnp.max(x, axis=0, keepdims=True))
    out_ref[...] = (e / jnp.sum(e, axis=0, keepdims=True)).T


_smax_call = pl.pallas_call(
    _smax_body,
    out_shape=jax.ShapeDtypeStruct((BATCH, N_COMMS), jnp.float32),
)


_RC = 10
_NCH = SEQ_LEN // _RC


def _sc_body(tab_hbm, mm_hbm, out_hbm, shard_v, ib0, ib1, out_v, sem0, sem1):
    t = lax.axis_index("s") * _NC + lax.axis_index("c")
    pltpu.sync_copy(tab_hbm.at[t], shard_v)

    z = jnp.zeros((_LANES,), jnp.int32)

    @pl.loop(0, _NG)
    def _zero(g):
        out_v[0, pl.ds(g * _LANES, _LANES)] = z
        out_v[1, pl.ds(g * _LANES, _LANES)] = z

    ibs = (ib0, ib1)
    sems = (sem0, sem1)

    def _fire(c, slot):
        pltpu.async_copy(mm_hbm.at[pl.ds(c * _RC, _RC), :], ibs[slot],
                         sems[slot])

    def _process(c, slot):
        ib = ibs[slot]
        pltpu.make_async_copy(mm_hbm.at[pl.ds(c * _RC, _RC), :], ib,
                              sems[slot]).wait()

        @plsc.parallel_loop(0, _NG, unroll=16)
        def _grp(g):
            a0, a1 = z, z
            for r in range(_RC):
                pv = plsc.load_gather(shard_v, [ib[r, pl.ds(g * _LANES, _LANES)]])
                a0 = a0 + ((pv << 16) >> 16)
                a1 = a1 + (pv >> 16)
            plsc.addupdate(out_v.at[0, pl.ds(g * _LANES, _LANES)], a0)
            plsc.addupdate(out_v.at[1, pl.ds(g * _LANES, _LANES)], a1)

    _fire(0, 0)

    @pl.loop(0, _NCH, step=2)
    def _chunk2(c0):
        _fire(c0 + 1, 1)
        _process(c0, 0)

        @pl.when(c0 + 2 < _NCH)
        def _():
            _fire(c0 + 2, 0)

        _process(c0 + 1, 1)

    pltpu.sync_copy(out_v.at[0], out_hbm.at[t])
    pltpu.sync_copy(out_v.at[1], out_hbm.at[t + _NW])


@functools.cache
def _make_sc_call():
    return functools.partial(
        pl.kernel,
        out_type=jax.ShapeDtypeStruct((N_COMMS, BATCH), jnp.int32),
        mesh=plsc.VectorSubcoreMesh(
            core_axis_name="c", subcore_axis_name="s", num_cores=_NC, num_subcores=_NS
        ),
        compiler_params=pltpu.CompilerParams(
            needs_layout_passes=False, use_tc_tiling_on_sc=False
        ),
        scratch_types=[
            pltpu.VMEM((_VW,), jnp.int32),
            pltpu.VMEM((_RC, BATCH), jnp.int32),
            pltpu.VMEM((_RC, BATCH), jnp.int32),
            pltpu.VMEM((2, BATCH), jnp.int32),
            pltpu.SemaphoreType.DMA,
            pltpu.SemaphoreType.DMA,
        ],
    )(_sc_body)


def kernel(m, m_lens, unigram_freq, comm_N):
    packed, mm = _tab_call(unigram_freq, unigram_freq,
                           comm_N.reshape(N_COMMS, 1), m,
                           m_lens.reshape(1, BATCH))
    nll = _make_sc_call()(packed, mm)
    return _smax_call(nll)
